# baseline (speedup 1.0000x reference)
import functools

import jax
import jax.numpy as jnp
import numpy as np
from jax import lax
from jax.experimental import pallas as pl
from jax.experimental.pallas import tpu as pltpu
from jax.experimental.pallas import tpu_sc as plsc

_B = 4096
_D = 64
_NS = 26
_NSP = 32
_HS = _NSP // 2
_V = 4823
_NSPLIT = 2
_BH = _B // _NSPLIT

_NC = 2
_NSC = 16
_NW = _NC * _NSC
_CH = 128

_sc_mesh = plsc.VectorSubcoreMesh(core_axis_name="c", subcore_axis_name="s")


def _make_sc_gather(nb):
    spw = nb // _NW
    rpw = spw * _NSP
    nch = rpw // _CH
    assert rpw % _CH == 0

    @functools.partial(
        pl.kernel,
        mesh=_sc_mesh,
        compiler_params=pltpu.CompilerParams(use_tc_tiling_on_sc=False),
        out_type=jax.ShapeDtypeStruct((nb * _NSP, _D), jnp.float32),
        scratch_types=[
            pltpu.VMEM((nch, _CH), jnp.int32),
            pltpu.VMEM((_CH, _D), jnp.float32),
            pltpu.VMEM((_CH, _D), jnp.float32),
            pltpu.SemaphoreType.DMA,
            pltpu.SemaphoreType.DMA,
            pltpu.SemaphoreType.DMA,
            pltpu.SemaphoreType.DMA,
        ],
    )
    def sc_gather(idx_hbm, table_hbm, out_hbm, idx_v, buf0, buf1, g0, g1, s0, s1):
        wid = lax.axis_index("s") * _NC + lax.axis_index("c")
        rbase = wid * rpw
        pltpu.sync_copy(idx_hbm.at[wid], idx_v)

        def body(h, carry):
            j0 = 2 * h
            j1 = 2 * h + 1
            c0 = pltpu.async_copy(table_hbm.at[idx_v.at[j0]], buf0, g0)
            c1 = pltpu.async_copy(table_hbm.at[idx_v.at[j1]], buf1, g1)
            c0.wait()
            w0 = pltpu.async_copy(
                buf0, out_hbm.at[pl.ds(rbase + j0 * _CH, _CH)], s0)
            c1.wait()
            w1 = pltpu.async_copy(
                buf1, out_hbm.at[pl.ds(rbase + j1 * _CH, _CH)], s1)
            w0.wait()
            w1.wait()
            return carry

        lax.fori_loop(0, nch // 2, body, 0)

    return sc_gather


_sc_gather_half = _make_sc_gather(_BH)

_BBLK = 256
_NBLK = _BH // _BBLK


def _tc_body(dense, y2p, wb1, wb2, wb3, wb4, wt1a, w22f, wt2, wt3, out):
    f32 = jnp.float32
    x = dense[:]
    y1 = jnp.maximum(jnp.dot(x, wb1[:], preferred_element_type=f32), 0.0)
    y1 = jnp.maximum(jnp.dot(y1, wb2[:], preferred_element_type=f32), 0.0)
    y1 = jnp.maximum(jnp.dot(y1, wb3[:], preferred_element_type=f32), 0.0)
    y1 = jnp.dot(y1, wb4[:], preferred_element_type=f32)

    t5 = y2p[:].reshape(_BBLK, _HS, 128)
    h0 = t5[:, :, :_D]
    h1 = t5[:, :, _D:]
    rid = lax.broadcasted_iota(jnp.int32, (_BBLK, _HS, _D), 1)
    y1b = lax.broadcast_in_dim(y1, (_BBLK, _HS, _D), (0, 2))
    h0 = jnp.where(rid == _NS // 2, y1b, h0)

    bdot = lambda a, b: lax.dot_general(
        a, b, dimension_numbers=(((2,), (2,)), ((0,), (0,))),
        preferred_element_type=f32)
    z00 = bdot(h0, h0).reshape(_BBLK, _HS * _HS)
    z01 = bdot(h0, h1).reshape(_BBLK, _HS * _HS)
    z11 = bdot(h1, h1).reshape(_BBLK, _HS * _HS)
    zf = jnp.concatenate([z00, z01, z11], axis=1)

    h = jnp.dot(y1, wt1a[:], preferred_element_type=f32)
    h = h + jnp.dot(zf, w22f[:], preferred_element_type=f32)
    h = jnp.maximum(h, 0.0)
    h = jnp.maximum(jnp.dot(h, wt2[:], preferred_element_type=f32), 0.0)
    out[:] = jax.nn.sigmoid(jnp.dot(h, wt3[:], preferred_element_type=f32))


def _const_spec(shape):
    return pl.BlockSpec(shape, lambda b: (0,) * len(shape))


_tc_call = pl.pallas_call(
    _tc_body,
    grid=(_NBLK,),
    in_specs=[
        pl.BlockSpec((_BBLK, 13), lambda b: (b, 0)),
        pl.BlockSpec((_BBLK * _HS, 128), lambda b: (b, 0)),
        _const_spec((13, 512)),
        _const_spec((512, 256)),
        _const_spec((256, 64)),
        _const_spec((64, _D)),
        _const_spec((_D, 512)),
        _const_spec((3 * _HS * _HS, 512)),
        _const_spec((512, 256)),
        _const_spec((256, 1)),
    ],
    out_specs=pl.BlockSpec((_BBLK, 1), lambda b: (b, 0)),
    out_shape=jax.ShapeDtypeStruct((_BH, 1), jnp.float32),
)


def _pair_idx(i, j):
    return i * (i - 1) // 2 + j


def _prep_weights(Wt1):
    wt1a = Wt1[:_D]
    wz = Wt1[_D:]
    q = _HS * _HS
    pos, rows = [], []

    def slot_col(a, b):
        ka, pa = divmod(a, 2)
        kb, pb = divmod(b, 2)
        if pa == 0 and pb == 0:
            return ka * _HS + kb
        if pa == 1 and pb == 1:
            return 2 * q + ka * _HS + kb
        if pa == 0 and pb == 1:
            return q + ka * _HS + kb
        return q + kb * _HS + ka

    for a in range(_NS):
        for b in range(a):
            pos.append(slot_col(a, b))
            rows.append(_pair_idx(a + 1, b + 1))
    for s in range(_NS):
        pos.append(slot_col(s, _NS))
        rows.append(_pair_idx(s + 1, 0))
    w22f = jnp.zeros((3 * q, 512), jnp.float32).at[
        jnp.asarray(pos, dtype=jnp.int32)].set(wz[jnp.asarray(rows)])
    return wt1a, w22f


def kernel(dense_input, sparse_input, emb, Wb1, Wb2, Wb3, Wb4, Wt1, Wt2, Wt3):
    wt1a, w22f = _prep_weights(Wt1)

    idx = sparse_input.astype(jnp.int32)
    idxp = jnp.concatenate(
        [idx, jnp.zeros((_B, _NSP - _NS), jnp.int32)], axis=1).reshape(-1)
    outs = []
    for h in range(_NSPLIT):
        idx_h = idxp[h * _BH * _NSP:(h + 1) * _BH * _NSP].reshape(_NW, -1, _CH)
        y2p_h = _sc_gather_half(idx_h, emb)
        y2v = y2p_h.reshape(_BH * _HS, 2 * _D)
        dense_h = dense_input[h * _BH:(h + 1) * _BH]
        outs.append(_tc_call(dense_h, y2v, Wb1, Wb2, Wb3, Wb4,
                             wt1a, w22f, Wt2, Wt3))
    return jnp.concatenate(outs, axis=0)


# speedup vs baseline: 5.6760x; 5.6760x over previous
import functools

import jax
import jax.numpy as jnp
import numpy as np
from jax import lax
from jax.experimental import pallas as pl
from jax.experimental.pallas import tpu as pltpu
from jax.experimental.pallas import tpu_sc as plsc

_B = 4096
_D = 64
_NS = 26
_NSP = 32
_HS = _NSP // 2
_V = 4823
_NSPLIT = 2
_BH = _B // _NSPLIT

_NC = 2
_NSC = 16
_NW = _NC * _NSC
_CH = 128

_sc_mesh = plsc.VectorSubcoreMesh(core_axis_name="c", subcore_axis_name="s")


def _make_sc_gather(nb):
    spw = nb // _NW
    rpw = spw * _NSP
    nch = rpw // _CH
    assert rpw % _CH == 0

    @functools.partial(
        pl.kernel,
        mesh=_sc_mesh,
        compiler_params=pltpu.CompilerParams(use_tc_tiling_on_sc=False),
        out_type=jax.ShapeDtypeStruct((nb * _NSP, _D), jnp.float32),
        scratch_types=[
            pltpu.VMEM((nch, _CH), jnp.int32),
            pltpu.VMEM((_CH, _D), jnp.float32),
            pltpu.VMEM((_CH, _D), jnp.float32),
            pltpu.SemaphoreType.DMA,
            pltpu.SemaphoreType.DMA,
            pltpu.SemaphoreType.DMA,
            pltpu.SemaphoreType.DMA,
        ],
    )
    def sc_gather(idx_hbm, table_hbm, out_hbm, idx_v, buf0, buf1, g0, g1, s0, s1):
        wid = lax.axis_index("s") * _NC + lax.axis_index("c")
        rbase = wid * rpw
        pltpu.sync_copy(idx_hbm.at[wid], idx_v)

        def body(h, carry):
            j0 = 2 * h
            j1 = 2 * h + 1
            c0 = pltpu.async_copy(table_hbm.at[idx_v.at[j0]], buf0, g0)
            c1 = pltpu.async_copy(table_hbm.at[idx_v.at[j1]], buf1, g1)
            c0.wait()
            w0 = pltpu.async_copy(
                buf0, out_hbm.at[pl.ds(rbase + j0 * _CH, _CH)], s0)
            c1.wait()
            w1 = pltpu.async_copy(
                buf1, out_hbm.at[pl.ds(rbase + j1 * _CH, _CH)], s1)
            w0.wait()
            w1.wait()
            return carry

        lax.fori_loop(0, nch // 2, body, 0)

    return sc_gather


_sc_gather_half = _make_sc_gather(_BH)

_BBLK = 256
_NBLK = _BH // _BBLK


def _tc_body(dense, y2p, wb1, wb2, wb3, wb4, wt1a, w22f, wt2, wt3, out):
    f32 = jnp.float32
    x = dense[:]
    y1 = jnp.maximum(jnp.dot(x, wb1[:], preferred_element_type=f32), 0.0)
    y1 = jnp.maximum(jnp.dot(y1, wb2[:], preferred_element_type=f32), 0.0)
    y1 = jnp.maximum(jnp.dot(y1, wb3[:], preferred_element_type=f32), 0.0)
    y1 = jnp.dot(y1, wb4[:], preferred_element_type=f32)

    t5 = y2p[:].reshape(_BBLK, _HS, 128)
    h0 = t5[:, :, :_D]
    h1 = t5[:, :, _D:]
    rid = lax.broadcasted_iota(jnp.int32, (_BBLK, _HS, _D), 1)
    y1b = lax.broadcast_in_dim(y1, (_BBLK, _HS, _D), (0, 2))
    h0 = jnp.where(rid == _NS // 2, y1b, h0)

    bdot = lambda a, b: lax.dot_general(
        a, b, dimension_numbers=(((2,), (2,)), ((0,), (0,))),
        preferred_element_type=f32)
    z00 = bdot(h0, h0).reshape(_BBLK, _HS * _HS)
    z01 = bdot(h0, h1).reshape(_BBLK, _HS * _HS)
    z11 = bdot(h1, h1).reshape(_BBLK, _HS * _HS)
    zf = jnp.concatenate([z00, z01, z11], axis=1)

    h = jnp.dot(y1, wt1a[:], preferred_element_type=f32)
    h = h + jnp.dot(zf, w22f[:], preferred_element_type=f32)
    h = jnp.maximum(h, 0.0)
    h = jnp.maximum(jnp.dot(h, wt2[:], preferred_element_type=f32), 0.0)
    out[:] = jax.nn.sigmoid(jnp.dot(h, wt3[:], preferred_element_type=f32))


def _const_spec(shape):
    return pl.BlockSpec(shape, lambda b: (0,) * len(shape))


_tc_call = pl.pallas_call(
    _tc_body,
    grid=(_NBLK,),
    in_specs=[
        pl.BlockSpec((_BBLK, 13), lambda b: (b, 0)),
        pl.BlockSpec((_BBLK * _HS, 128), lambda b: (b, 0)),
        _const_spec((13, 512)),
        _const_spec((512, 256)),
        _const_spec((256, 64)),
        _const_spec((64, _D)),
        _const_spec((_D, 512)),
        _const_spec((3 * _HS * _HS, 512)),
        _const_spec((512, 256)),
        _const_spec((256, 1)),
    ],
    out_specs=pl.BlockSpec((_BBLK, 1), lambda b: (b, 0)),
    out_shape=jax.ShapeDtypeStruct((_BH, 1), jnp.float32),
)


def _pair_idx(i, j):
    return i * (i - 1) // 2 + j


def _prep_weights(Wt1):
    wt1a = Wt1[:_D]
    wz = Wt1[_D:]
    q = _HS * _HS
    pos, rows = [], []

    def slot_col(a, b):
        ka, pa = divmod(a, 2)
        kb, pb = divmod(b, 2)
        if pa == 0 and pb == 0:
            return ka * _HS + kb
        if pa == 1 and pb == 1:
            return 2 * q + ka * _HS + kb
        if pa == 0 and pb == 1:
            return q + ka * _HS + kb
        return q + kb * _HS + ka

    for a in range(_NS):
        for b in range(a):
            pos.append(slot_col(a, b))
            rows.append(_pair_idx(a + 1, b + 1))
    for s in range(_NS):
        pos.append(slot_col(s, _NS))
        rows.append(_pair_idx(s + 1, 0))
    w22f = jnp.zeros((3 * q, 512), jnp.float32).at[
        jnp.asarray(pos, dtype=jnp.int32)].set(wz[jnp.asarray(rows)])
    return wt1a, w22f


def kernel(dense_input, sparse_input, emb, Wb1, Wb2, Wb3, Wb4, Wt1, Wt2, Wt3):
    wt1a, w22f = _prep_weights(Wt1)

    idx = sparse_input.astype(jnp.int32)
    idxp = jnp.concatenate([idx, idx[:, :_NSP - _NS]], axis=1).reshape(-1)
    outs = []
    for h in range(_NSPLIT):
        idx_h = idxp[h * _BH * _NSP:(h + 1) * _BH * _NSP].reshape(_NW, -1, _CH)
        y2p_h = _sc_gather_half(idx_h, emb)
        y2v = y2p_h.reshape(_BH * _HS, 2 * _D)
        dense_h = dense_input[h * _BH:(h + 1) * _BH]
        outs.append(_tc_call(dense_h, y2v, Wb1, Wb2, Wb3, Wb4,
                             wt1a, w22f, Wt2, Wt3))
    return jnp.concatenate(outs, axis=0)
